# baseline (speedup 1.0000x reference)
import functools

import jax
import jax.numpy as jnp
from jax import lax
from jax.experimental import pallas as pl
from jax.experimental.pallas import tpu as pltpu
from jax.experimental.pallas import tpu_sc as plsc


def _sc_gather(table, idx, b, d):
    info = plsc.get_sparse_core_info()
    nc, ns = info.num_cores, info.num_subcores
    b_per_w = 8
    n_active = b // b_per_w
    mesh = plsc.VectorSubcoreMesh(core_axis_name="c", subcore_axis_name="s")

    @functools.partial(
        pl.kernel,
        mesh=mesh,
        out_type=jax.ShapeDtypeStruct((b, d), jnp.float32),
        scratch_types=[
            pltpu.VMEM((b_per_w,), jnp.int32),
            pltpu.VMEM((b_per_w, d), jnp.float32),
            pltpu.SemaphoreType.DMA,
        ],
    )
    def gather_kernel(table_hbm, idx_hbm, out_hbm, idx_v, rows_v, sem):
        wid = lax.axis_index("s") * nc + lax.axis_index("c")

        @pl.when(wid < n_active)
        def _():
            base = wid * b_per_w
            pltpu.sync_copy(idx_hbm.at[pl.ds(base, b_per_w)], idx_v)
            pltpu.async_copy(table_hbm.at[idx_v], rows_v, sem).wait()
            pltpu.sync_copy(rows_v, out_hbm.at[pl.ds(base, b_per_w)])

    return gather_kernel(table, idx)


def _add_body(x_ref, g_ref, o_ref):
    o_ref[...] = x_ref[...] + g_ref[...]


def kernel(x, time, embeddings):
    b, d, h, w = x.shape
    hw = h * w
    t32 = time.astype(jnp.int32)

    gathered = _sc_gather(embeddings[:, :d], t32, b, d)

    rows = b * d
    rb = 2048
    x2 = x.reshape(rows, hw)
    g2 = gathered.reshape(rows, 1)
    out = pl.pallas_call(
        _add_body,
        grid=(rows // rb,),
        in_specs=[
            pl.BlockSpec((rb, hw), lambda i: (i, 0)),
            pl.BlockSpec((rb, 1), lambda i: (i, 0)),
        ],
        out_specs=pl.BlockSpec((rb, hw), lambda i: (i, 0)),
        out_shape=jax.ShapeDtypeStruct((rows, hw), x.dtype),
    )(x2, g2)
    return out.reshape(b, d, h, w)


# speedup vs baseline: 2.1139x; 2.1139x over previous
import functools

import jax
import jax.numpy as jnp
from jax import lax
from jax.experimental import pallas as pl
from jax.experimental.pallas import tpu as pltpu
from jax.experimental.pallas import tpu_sc as plsc


def _sc_gather(table, idx, b, d):
    info = plsc.get_sparse_core_info()
    nc = info.num_cores
    b_per_w = 8
    n_active = b // b_per_w
    mesh = plsc.VectorSubcoreMesh(core_axis_name="c", subcore_axis_name="s")

    @functools.partial(
        pl.kernel,
        mesh=mesh,
        out_type=jax.ShapeDtypeStruct((b, d), jnp.float32),
        scratch_types=[
            pltpu.VMEM((b_per_w,), jnp.int32),
            pltpu.VMEM((b_per_w, d), jnp.float32),
            pltpu.SemaphoreType.DMA,
        ],
    )
    def gather_kernel(table_hbm, idx_hbm, out_hbm, idx_v, rows_v, sem):
        wid = lax.axis_index("s") * nc + lax.axis_index("c")

        @pl.when(wid < n_active)
        def _():
            base = wid * b_per_w
            pltpu.sync_copy(idx_hbm.at[pl.ds(base, b_per_w)], idx_v)
            pltpu.async_copy(table_hbm.at[idx_v], rows_v, sem).wait()
            pltpu.sync_copy(rows_v, out_hbm.at[pl.ds(base, b_per_w)])

    return gather_kernel(table, idx)


def _make_add_body(b):
    def _add_body(x_ref, g_ref, o_ref):
        i = pl.program_id(0)
        onehot = (lax.broadcasted_iota(jnp.int32, (b, 1), 0) == i).astype(
            jnp.float32
        )
        col = lax.dot_general(
            g_ref[...], onehot, (((0,), (0,)), ((), ())),
            preferred_element_type=jnp.float32,
        )
        o_ref[0] = x_ref[0] + col

    return _add_body


def kernel(x, time, embeddings):
    b, d, h, w = x.shape
    hw = h * w
    t32 = time.astype(jnp.int32)
    xr = x.reshape(b, d, hw)

    gathered = _sc_gather(embeddings[:, :d], t32, b, d)

    out = pl.pallas_call(
        _make_add_body(b),
        grid=(b,),
        in_specs=[
            pl.BlockSpec((1, d, hw), lambda i: (i, 0, 0)),
            pl.BlockSpec((b, d), lambda i: (0, 0)),
        ],
        out_specs=pl.BlockSpec((1, d, hw), lambda i: (i, 0, 0)),
        out_shape=jax.ShapeDtypeStruct((b, d, hw), x.dtype),
    )(xr, gathered)
    return out.reshape(b, d, h, w)


# speedup vs baseline: 2.1968x; 1.0392x over previous
import functools

import jax
import jax.numpy as jnp
from jax import lax
from jax.experimental import pallas as pl
from jax.experimental.pallas import tpu as pltpu
from jax.experimental.pallas import tpu_sc as plsc


def _sc_gather(table, idx, b, d):
    info = plsc.get_sparse_core_info()
    nc = info.num_cores
    b_per_w = 8
    n_active = b // b_per_w
    mesh = plsc.VectorSubcoreMesh(core_axis_name="c", subcore_axis_name="s")

    @functools.partial(
        pl.kernel,
        mesh=mesh,
        out_type=jax.ShapeDtypeStruct((b, d), jnp.float32),
        scratch_types=[
            pltpu.VMEM((b_per_w,), jnp.int32),
            pltpu.VMEM((b_per_w, d), jnp.float32),
            pltpu.SemaphoreType.DMA,
        ],
    )
    def gather_kernel(table_hbm, idx_hbm, out_hbm, idx_v, rows_v, sem):
        wid = lax.axis_index("s") * nc + lax.axis_index("c")

        @pl.when(wid < n_active)
        def _():
            base = wid * b_per_w
            pltpu.sync_copy(idx_hbm.at[pl.ds(base, b_per_w)], idx_v)
            pltpu.async_copy(table_hbm.at[idx_v], rows_v, sem).wait()
            pltpu.sync_copy(rows_v, out_hbm.at[pl.ds(base, b_per_w)])

    return gather_kernel(table, idx)


def _make_add_body(b, bb):
    def _add_body(x_ref, g_ref, o_ref):
        i = pl.program_id(0)
        rows = lax.broadcasted_iota(jnp.int32, (b, bb), 0)
        sel = lax.broadcasted_iota(jnp.int32, (b, bb), 1) + i * bb
        onehot = (rows == sel).astype(jnp.float32)
        cols = lax.dot_general(
            g_ref[...], onehot, (((0,), (0,)), ((), ())),
            preferred_element_type=jnp.float32,
        )
        for j in range(bb):
            o_ref[j] = x_ref[j] + cols[:, j : j + 1]

    return _add_body


def kernel(x, time, embeddings):
    b, d, h, w = x.shape
    hw = h * w
    t32 = time.astype(jnp.int32)
    xr = x.reshape(b, d, hw)

    gathered = _sc_gather(embeddings[:, :d], t32, b, d)

    bb = 4
    out = pl.pallas_call(
        _make_add_body(b, bb),
        grid=(b // bb,),
        in_specs=[
            pl.BlockSpec((bb, d, hw), lambda i: (i, 0, 0)),
            pl.BlockSpec((b, d), lambda i: (0, 0)),
        ],
        out_specs=pl.BlockSpec((bb, d, hw), lambda i: (i, 0, 0)),
        out_shape=jax.ShapeDtypeStruct((b, d, hw), x.dtype),
    )(xr, gathered)
    return out.reshape(b, d, h, w)
